# Initial kernel scaffold; baseline (speedup 1.0000x reference)
import jax
import jax.numpy as jnp
from jax.experimental import pallas as pl


def kernel(x, token_embedding, added_embedding, vqgan_codebook, vqgan_proj_W):
    raise NotImplementedError("write your pallas kernel here")


# speedup vs baseline: 13.6606x; 13.6606x over previous
import functools

import jax
import jax.numpy as jnp
from jax import lax
from jax.experimental import pallas as pl
from jax.experimental.pallas import tpu as pltpu
from jax.experimental.pallas import tpu_sc as plsc

ATO = 100000
EMBED = 128
VQ_DIM = 256

BLK = 1024
TEXT_BLKS = 98
ADD_BLKS = 8
PROJ_BLKS = 16
TBL_BLKS = TEXT_BLKS + ADD_BLKS + PROJ_BLKS
TBL_ROWS = TBL_BLKS * BLK
SHIFT = TEXT_BLKS * BLK - ATO

NC, NS, L = 2, 16, 16
NW = NC * NS
NTOK = 1024 * 200
CHUNK = NTOK // NW
BATCH = 128
NB = CHUNK // BATCH


def _build_table_body(tok_ref, add_ref, cb_ref, w_ref, out_ref):
    g = pl.program_id(0)

    @pl.when(g < TEXT_BLKS)
    def _():
        out_ref[...] = tok_ref[...]

    @pl.when((g >= TEXT_BLKS) & (g < TEXT_BLKS + ADD_BLKS))
    def _():
        out_ref[...] = add_ref[...]

    @pl.when(g >= TEXT_BLKS + ADD_BLKS)
    def _():
        out_ref[...] = lax.dot_general(
            cb_ref[...], w_ref[...],
            dimension_numbers=(((1,), (1,)), ((), ())),
            preferred_element_type=jnp.float32,
        )


def _build_table(token_embedding, added_embedding, vqgan_codebook, vqgan_proj_W):
    return pl.pallas_call(
        _build_table_body,
        grid=(TBL_BLKS,),
        in_specs=[
            pl.BlockSpec((BLK, EMBED), lambda g: (jnp.minimum(g, TEXT_BLKS - 1), 0)),
            pl.BlockSpec((BLK, EMBED), lambda g: (jnp.clip(g - TEXT_BLKS, 0, ADD_BLKS - 1), 0)),
            pl.BlockSpec((BLK, VQ_DIM), lambda g: (jnp.clip(g - TEXT_BLKS - ADD_BLKS, 0, PROJ_BLKS - 1), 0)),
            pl.BlockSpec((EMBED, VQ_DIM), lambda g: (0, 0)),
        ],
        out_specs=pl.BlockSpec((BLK, EMBED), lambda g: (g, 0)),
        out_shape=jax.ShapeDtypeStruct((TBL_ROWS, EMBED), jnp.float32),
    )(token_embedding, added_embedding, vqgan_codebook, vqgan_proj_W)


@functools.cache
def _sc_gather_fn():
    mesh = plsc.VectorSubcoreMesh(
        core_axis_name="c", subcore_axis_name="s", num_cores=NC, num_subcores=NS)
    return functools.partial(
        pl.kernel,
        out_type=jax.ShapeDtypeStruct((NTOK, EMBED), jnp.float32),
        mesh=mesh,
        scratch_types=[
            pltpu.VMEM((NB, BATCH), jnp.int32),
            pltpu.VMEM((BATCH, EMBED), jnp.float32),
            pltpu.VMEM((BATCH, EMBED), jnp.float32),
            pltpu.SemaphoreType.DMA,
            pltpu.SemaphoreType.DMA,
            pltpu.SemaphoreType.DMA,
            pltpu.SemaphoreType.DMA,
        ],
    )(_sc_gather_body)


def _sc_gather_body(x_hbm, tbl_hbm, out_hbm, idx_v, buf0, buf1, gs0, gs1, os0, os1):
    wid = lax.axis_index("s") * NC + lax.axis_index("c")
    base = wid * CHUNK

    pltpu.sync_copy(x_hbm.at[wid], idx_v)

    bufs = (buf0, buf1)
    gsems = (gs0, gs1)
    osems = (os0, os1)

    def remap(k):
        def body(j, carry):
            v = idx_v[k, pl.ds(j * L, L)]
            idx_v[k, pl.ds(j * L, L)] = jnp.where(v >= ATO, v + SHIFT, v)
            return carry
        lax.fori_loop(0, BATCH // L, body, 0)

    def g_start(k, b):
        pltpu.make_async_copy(tbl_hbm.at[idx_v.at[k]], bufs[b], gsems[b]).start()

    def g_wait(k, b):
        pltpu.make_async_copy(tbl_hbm.at[idx_v.at[k]], bufs[b], gsems[b]).wait()

    def o_start(k, b):
        pltpu.make_async_copy(
            bufs[b], out_hbm.at[pl.ds(base + k * BATCH, BATCH)], osems[b]).start()

    def o_wait(k, b):
        pltpu.make_async_copy(
            bufs[b], out_hbm.at[pl.ds(base + k * BATCH, BATCH)], osems[b]).wait()

    remap(0)
    g_start(0, 0)
    remap(1)
    g_start(1, 1)

    def loop_body(i, carry):
        for b in range(2):
            k = 2 * i + b
            g_wait(k, b)
            o_start(k, b)
            nk = k + 2

            @pl.when(nk < NB)
            def _():
                remap(nk)
                o_wait(k, b)
                g_start(nk, b)
        return carry

    lax.fori_loop(0, NB // 2, loop_body, 0)

    o_wait(NB - 2, 0)
    o_wait(NB - 1, 1)


def kernel(x, token_embedding, added_embedding, vqgan_codebook, vqgan_proj_W):
    tbl = _build_table(token_embedding, added_embedding, vqgan_codebook, vqgan_proj_W)
    x_w = x.reshape(NW, NB, BATCH)
    out = _sc_gather_fn()(x_w, tbl)
    return out.reshape(x.shape[0], x.shape[1], EMBED)
